# Initial kernel scaffold; baseline (speedup 1.0000x reference)
import jax
import jax.numpy as jnp
from jax.experimental import pallas as pl


def kernel(vfts, adjs, rels, W1_0, b1_0, W1_2, b1_2, W2_0, b2_0, W2_2, b2_2):
    raise NotImplementedError("write your pallas kernel here")


# speedup vs baseline: 19.2133x; 19.2133x over previous
import functools

import jax
import jax.numpy as jnp
from jax import lax
from jax.experimental import pallas as pl
from jax.experimental.pallas import tpu as pltpu
from jax.experimental.pallas import tpu_sc as plsc

N_NODES = 10000
N_EDGES = 320000
D_IN = 128
D_OUT = 128
N_REL = 3

NC = 2
NS = 16
L = 16

HALF = D_IN // NC
SEGS = N_REL * N_NODES
EPT = N_EDGES // NS
CH = 80
NCHUNK = EPT // CH
ZSTRIPE = SEGS // NS


def _sc_aggregate(vhalves, adjs, rels, zeros):
    mesh = plsc.VectorSubcoreMesh(core_axis_name="c", subcore_axis_name="s")

    def body(vhalves_hbm, adjs_hbm, rel_hbm, zeros_hbm, out_hbm,
             shared, srcb, dstb, relb, sidxb, rows, semi, semg, sems):
        c = lax.axis_index("c")
        s = lax.axis_index("s")
        base = s * EPT
        vsrc = vhalves_hbm.at[c]

        def issue_idx(j, b):
            off = base + j * CH
            pltpu.async_copy(adjs_hbm.at[0, pl.ds(off, CH)], srcb.at[b],
                             semi.at[b])
            pltpu.async_copy(adjs_hbm.at[1, pl.ds(off, CH)], dstb.at[b],
                             semi.at[b])
            pltpu.async_copy(rel_hbm.at[pl.ds(off, CH)], relb.at[b],
                             semi.at[b])

        def wait_idx(j, b):
            off = base + j * CH
            pltpu.make_async_copy(adjs_hbm.at[0, pl.ds(off, CH)], srcb.at[b],
                                  semi.at[b]).wait()
            pltpu.make_async_copy(adjs_hbm.at[1, pl.ds(off, CH)], dstb.at[b],
                                  semi.at[b]).wait()
            pltpu.make_async_copy(rel_hbm.at[pl.ds(off, CH)], relb.at[b],
                                  semi.at[b]).wait()

        def issue_gather(b):
            pltpu.async_copy(vsrc.at[srcb.at[b]], rows.at[b], semg.at[b])

        def wait_gather(b):
            pltpu.make_async_copy(vsrc.at[srcb.at[b]], rows.at[b],
                                  semg.at[b]).wait()

        def wait_scatter(b):
            pltpu.make_async_copy(rows.at[b], shared.at[sidxb.at[b]],
                                  sems.at[b]).wait()

        pltpu.sync_copy(zeros_hbm.at[pl.ds(s * ZSTRIPE, ZSTRIPE)],
                        shared.at[pl.ds(s * ZSTRIPE, ZSTRIPE)])

        issue_idx(0, 0)
        wait_idx(0, 0)
        issue_gather(0)
        issue_idx(1, 1)

        plsc.subcore_barrier()

        @pl.loop(0, NCHUNK, step=2)
        def _edges(j0):
            for p in range(2):
                j = j0 + p
                n = 1 - p

                @pl.when(j + 1 < NCHUNK)
                def _start_next():
                    wait_idx(j + 1, n)

                    @pl.when(j >= 1)
                    def _recycle():
                        wait_scatter(n)

                    issue_gather(n)

                wait_gather(p)
                for k in range(CH // L):
                    dv = dstb[p, pl.ds(k * L, L)]
                    rv = relb[p, pl.ds(k * L, L)]
                    sidxb[p, pl.ds(k * L, L)] = rv * N_NODES + dv
                pltpu.async_copy(rows.at[p], shared.at[sidxb.at[p]],
                                 sems.at[p], add=True)

                @pl.when(j + 2 < NCHUNK)
                def _prefetch_idx():
                    issue_idx(j + 2, p)

        wait_scatter(0)
        wait_scatter(1)

        plsc.subcore_barrier()
        pltpu.sync_copy(shared.at[pl.ds(s * ZSTRIPE, ZSTRIPE)],
                        out_hbm.at[pl.ds(c * SEGS + s * ZSTRIPE, ZSTRIPE)])

    run = pl.kernel(
        body,
        out_type=jax.ShapeDtypeStruct((NC * SEGS, HALF), jnp.float32),
        mesh=mesh,
        scratch_types=[
            pltpu.VMEM_SHARED((SEGS, HALF), jnp.float32),
            pltpu.VMEM((2, CH), jnp.int32),
            pltpu.VMEM((2, CH), jnp.int32),
            pltpu.VMEM((2, CH), jnp.int32),
            pltpu.VMEM((2, CH), jnp.int32),
            pltpu.VMEM((2, CH, HALF), jnp.float32),
            pltpu.SemaphoreType.DMA((2,)),
            pltpu.SemaphoreType.DMA((2,)),
            pltpu.SemaphoreType.DMA((2,)),
        ],
        compiler_params=pltpu.CompilerParams(use_tc_tiling_on_sc=False),
        name="dssconv_sc_aggregate",
    )
    return run(vhalves, adjs, rels, zeros)


def _tc_split(vfts):
    TILE = 2000

    def body(x_ref, o_ref):
        o_ref[0] = x_ref[:, :HALF]
        o_ref[1] = x_ref[:, HALF:]

    return pl.pallas_call(
        body,
        grid=(N_NODES // TILE,),
        in_specs=[pl.BlockSpec((TILE, D_IN), lambda i: (i, 0))],
        out_specs=pl.BlockSpec((NC, TILE, HALF), lambda i: (0, i, 0)),
        out_shape=jax.ShapeDtypeStruct((NC, N_NODES, HALF), jnp.float32),
    )(vfts)


MTILE = 1000


def _tc_body(x_ref, a00_ref, a01_ref, a02_ref, a10_ref, a11_ref, a12_ref,
             w10_ref, b10_ref, w12_ref, b12_ref,
             w20_ref, b20_ref, w22_ref, b22_ref, out_ref):
    x = x_ref[...]
    lo = (a00_ref, a01_ref, a02_ref)
    hi = (a10_ref, a11_ref, a12_ref)
    aggs = [jnp.concatenate([lo[r][...], hi[r][...]], axis=1)
            for r in range(N_REL)]
    agg_all = aggs[0] + aggs[1] + aggs[2]
    w10, b10 = w10_ref[...], b10_ref[...]
    w12, b12 = w12_ref[...], b12_ref[...]
    w20, b20 = w20_ref[...], b20_ref[...]
    w22, b22 = w22_ref[...], b22_ref[...]
    for r in range(N_REL):
        h1 = jnp.maximum(jnp.dot(x + aggs[r], w10) + b10, 0.0)
        o1 = jnp.dot(h1, w12) + b12
        h2 = jnp.maximum(jnp.dot(x + (agg_all - aggs[r]), w20) + b20, 0.0)
        o2 = jnp.dot(h2, w22) + b22
        out_ref[:, r, :] = o1 + o2


def _tc_mlp(vfts, aggflat, W1_0, b1_0, W1_2, b1_2, W2_0, b2_0, W2_2, b2_2):
    nb = N_NODES // MTILE
    grid = (nb,)

    def aspec(c, r):
        off = (c * SEGS + r * N_NODES) // MTILE
        return pl.BlockSpec((MTILE, HALF), lambda i, off=off: (off + i, 0))

    full = lambda shape: pl.BlockSpec(shape, lambda i: (0,) * len(shape))
    return pl.pallas_call(
        _tc_body,
        grid=grid,
        in_specs=[
            pl.BlockSpec((MTILE, D_IN), lambda i: (i, 0)),
            aspec(0, 0), aspec(0, 1), aspec(0, 2),
            aspec(1, 0), aspec(1, 1), aspec(1, 2),
            full((D_IN, D_OUT)), full((1, D_OUT)),
            full((D_OUT, D_OUT)), full((1, D_OUT)),
            full((D_IN, D_OUT)), full((1, D_OUT)),
            full((D_OUT, D_OUT)), full((1, D_OUT)),
        ],
        out_specs=pl.BlockSpec((MTILE, N_REL, D_OUT), lambda i: (i, 0, 0)),
        out_shape=jax.ShapeDtypeStruct((N_NODES, N_REL, D_OUT), jnp.float32),
    )(vfts, aggflat, aggflat, aggflat, aggflat, aggflat, aggflat,
      W1_0, b1_0.reshape(1, -1), W1_2, b1_2.reshape(1, -1),
      W2_0, b2_0.reshape(1, -1), W2_2, b2_2.reshape(1, -1))


def kernel(vfts, adjs, rels, W1_0, b1_0, W1_2, b1_2, W2_0, b2_0, W2_2, b2_2):
    vhalves = _tc_split(vfts)
    zeros = jnp.zeros((SEGS, HALF), jnp.float32)
    aggflat = _sc_aggregate(vhalves, adjs.astype(jnp.int32),
                            rels.astype(jnp.int32), zeros)
    return _tc_mlp(vfts, aggflat, W1_0, b1_0, W1_2, b1_2,
                   W2_0, b2_0, W2_2, b2_2)


# speedup vs baseline: 23.3632x; 1.2160x over previous
import functools

import jax
import jax.numpy as jnp
from jax import lax
from jax.experimental import pallas as pl
from jax.experimental.pallas import tpu as pltpu
from jax.experimental.pallas import tpu_sc as plsc

N_NODES = 10000
N_EDGES = 320000
D_IN = 128
D_OUT = 128
N_REL = 3

NC = 2
NS = 16
L = 16

HALF = D_IN // NC
SEGS = N_REL * N_NODES
EPT = N_EDGES // NS
CH = 80
NCHUNK = EPT // CH
ZSTRIPE = SEGS // NS


def _sc_aggregate(vhalves, adjs, rels, zeros):
    mesh = plsc.VectorSubcoreMesh(core_axis_name="c", subcore_axis_name="s")

    def body(vhalves_hbm, adjs_hbm, rel_hbm, zeros_hbm, out_hbm,
             shared, srcb, dstb, relb, sidxb, rows, semi, semg, sems):
        c = lax.axis_index("c")
        s = lax.axis_index("s")
        base = s * EPT
        vsrc = vhalves_hbm.at[c]

        def issue_idx(j, b):
            off = base + j * CH
            pltpu.async_copy(adjs_hbm.at[0, pl.ds(off, CH)], srcb.at[b],
                             semi.at[b])
            pltpu.async_copy(adjs_hbm.at[1, pl.ds(off, CH)], dstb.at[b],
                             semi.at[b])
            pltpu.async_copy(rel_hbm.at[pl.ds(off, CH)], relb.at[b],
                             semi.at[b])

        def wait_idx(j, b):
            off = base + j * CH
            pltpu.make_async_copy(adjs_hbm.at[0, pl.ds(off, CH)], srcb.at[b],
                                  semi.at[b]).wait()
            pltpu.make_async_copy(adjs_hbm.at[1, pl.ds(off, CH)], dstb.at[b],
                                  semi.at[b]).wait()
            pltpu.make_async_copy(rel_hbm.at[pl.ds(off, CH)], relb.at[b],
                                  semi.at[b]).wait()

        def issue_gather(b):
            pltpu.async_copy(vsrc.at[srcb.at[b]], rows.at[b], semg.at[b])

        def wait_gather(b):
            pltpu.make_async_copy(vsrc.at[srcb.at[b]], rows.at[b],
                                  semg.at[b]).wait()

        def wait_scatter(b):
            pltpu.make_async_copy(rows.at[b], shared.at[sidxb.at[b]],
                                  sems.at[b]).wait()

        pltpu.sync_copy(zeros_hbm.at[pl.ds(s * ZSTRIPE, ZSTRIPE)],
                        shared.at[pl.ds(s * ZSTRIPE, ZSTRIPE)])

        issue_idx(0, 0)
        wait_idx(0, 0)
        issue_gather(0)
        issue_idx(1, 1)

        plsc.subcore_barrier()

        @pl.loop(0, NCHUNK, step=2)
        def _edges(j0):
            for p in range(2):
                j = j0 + p
                n = 1 - p

                @pl.when(j + 1 < NCHUNK)
                def _start_next():
                    wait_idx(j + 1, n)

                    @pl.when(j >= 1)
                    def _recycle():
                        wait_scatter(n)

                    issue_gather(n)

                wait_gather(p)
                for k in range(CH // L):
                    dv = dstb[p, pl.ds(k * L, L)]
                    rv = relb[p, pl.ds(k * L, L)]
                    sidxb[p, pl.ds(k * L, L)] = rv * N_NODES + dv
                pltpu.async_copy(rows.at[p], shared.at[sidxb.at[p]],
                                 sems.at[p], add=True)

                @pl.when(j + 2 < NCHUNK)
                def _prefetch_idx():
                    issue_idx(j + 2, p)

        wait_scatter(0)
        wait_scatter(1)

        plsc.subcore_barrier()
        pltpu.sync_copy(
            shared.at[pl.ds(s * ZSTRIPE, ZSTRIPE)],
            out_hbm.at[pl.ds(s * ZSTRIPE, ZSTRIPE), pl.ds(c * HALF, HALF)])

    run = pl.kernel(
        body,
        out_type=jax.ShapeDtypeStruct((SEGS, D_IN), jnp.float32),
        mesh=mesh,
        scratch_types=[
            pltpu.VMEM_SHARED((SEGS, HALF), jnp.float32),
            pltpu.VMEM((2, CH), jnp.int32),
            pltpu.VMEM((2, CH), jnp.int32),
            pltpu.VMEM((2, CH), jnp.int32),
            pltpu.VMEM((2, CH), jnp.int32),
            pltpu.VMEM((2, CH, HALF), jnp.float32),
            pltpu.SemaphoreType.DMA((2,)),
            pltpu.SemaphoreType.DMA((2,)),
            pltpu.SemaphoreType.DMA((2,)),
        ],
        compiler_params=pltpu.CompilerParams(use_tc_tiling_on_sc=False),
        name="dssconv_sc_aggregate",
    )
    return run(vhalves, adjs, rels, zeros)


def _tc_prep(vfts):
    TILE = 2000
    ZB = (SEGS // 2) // (N_NODES // TILE)

    def body(x_ref, o_ref, z_ref):
        xr = x_ref[...].reshape(TILE // 2, 2 * D_IN)
        o_ref[0] = jnp.concatenate(
            [xr[:, 0:HALF], xr[:, D_IN:D_IN + HALF]], axis=1)
        o_ref[1] = jnp.concatenate(
            [xr[:, HALF:D_IN], xr[:, D_IN + HALF:2 * D_IN]], axis=1)
        z_ref[...] = jnp.zeros((ZB, D_IN), jnp.float32)

    vh, zimg = pl.pallas_call(
        body,
        grid=(N_NODES // TILE,),
        in_specs=[pl.BlockSpec((TILE, D_IN), lambda i: (i, 0))],
        out_specs=[
            pl.BlockSpec((NC, TILE // 2, D_IN), lambda i: (0, i, 0)),
            pl.BlockSpec((ZB, D_IN), lambda i: (i, 0)),
        ],
        out_shape=[
            jax.ShapeDtypeStruct((NC, N_NODES // 2, D_IN), jnp.float32),
            jax.ShapeDtypeStruct((SEGS // 2, D_IN), jnp.float32),
        ],
    )(vfts)
    return vh.reshape(NC, N_NODES, HALF), zimg.reshape(SEGS, HALF)


MTILE = 2000


def _tc_body(x_ref, a0_ref, a1_ref, a2_ref,
             w10_ref, b10_ref, w12_ref, b12_ref,
             w20_ref, b20_ref, w22_ref, b22_ref, out_ref):
    x = x_ref[...]
    aggs = (a0_ref[...], a1_ref[...], a2_ref[...])
    agg_all = aggs[0] + aggs[1] + aggs[2]
    bf = jnp.bfloat16
    mm = lambda a, b: jnp.dot(a.astype(bf), b.astype(bf),
                              preferred_element_type=jnp.float32)
    w10, b10 = w10_ref[...], b10_ref[...]
    w12, b12 = w12_ref[...], b12_ref[...]
    w20, b20 = w20_ref[...], b20_ref[...]
    w22, b22 = w22_ref[...], b22_ref[...]
    for r in range(N_REL):
        h1 = jnp.maximum(mm(x + aggs[r], w10) + b10, 0.0)
        o1 = mm(h1, w12) + b12
        h2 = jnp.maximum(mm(x + (agg_all - aggs[r]), w20) + b20, 0.0)
        o2 = mm(h2, w22) + b22
        out_ref[r] = o1 + o2


def _tc_mlp(vfts, agg, W1_0, b1_0, W1_2, b1_2, W2_0, b2_0, W2_2, b2_2):
    nb = N_NODES // MTILE

    def aspec(r):
        off = r * nb
        return pl.BlockSpec((MTILE, D_IN), lambda i, off=off: (off + i, 0))

    full = lambda shape: pl.BlockSpec(shape, lambda i: (0,) * len(shape))
    out = pl.pallas_call(
        _tc_body,
        grid=(nb,),
        in_specs=[
            pl.BlockSpec((MTILE, D_IN), lambda i: (i, 0)),
            aspec(0), aspec(1), aspec(2),
            full((D_IN, D_OUT)), full((1, D_OUT)),
            full((D_OUT, D_OUT)), full((1, D_OUT)),
            full((D_IN, D_OUT)), full((1, D_OUT)),
            full((D_OUT, D_OUT)), full((1, D_OUT)),
        ],
        out_specs=pl.BlockSpec((N_REL, MTILE, D_OUT), lambda i: (0, i, 0)),
        out_shape=jax.ShapeDtypeStruct((N_REL, N_NODES, D_OUT), jnp.float32),
    )(vfts, agg, agg, agg,
      W1_0, b1_0.reshape(1, -1), W1_2, b1_2.reshape(1, -1),
      W2_0, b2_0.reshape(1, -1), W2_2, b2_2.reshape(1, -1))
    return jnp.transpose(out, (1, 0, 2))


def kernel(vfts, adjs, rels, W1_0, b1_0, W1_2, b1_2, W2_0, b2_0, W2_2, b2_2):
    vhalves, zeros = _tc_prep(vfts)
    agg = _sc_aggregate(vhalves, adjs.astype(jnp.int32),
                        rels.astype(jnp.int32), zeros)
    return _tc_mlp(vfts, agg, W1_0, b1_0, W1_2, b1_2,
                   W2_0, b2_0, W2_2, b2_2)


# speedup vs baseline: 23.8614x; 1.0213x over previous
import functools

import jax
import jax.numpy as jnp
from jax import lax
from jax.experimental import pallas as pl
from jax.experimental.pallas import tpu as pltpu
from jax.experimental.pallas import tpu_sc as plsc

N_NODES = 10000
N_EDGES = 320000
D_IN = 128
D_OUT = 128
N_REL = 3

NC = 2
NS = 16
L = 16

HALF = D_IN // NC
SEGS = N_REL * N_NODES
EPT = N_EDGES // NS
CH = 80
NCHUNK = EPT // CH
ZSTRIPE = SEGS // NS


def _sc_aggregate(vhalves, adjs, rels, zeros):
    mesh = plsc.VectorSubcoreMesh(core_axis_name="c", subcore_axis_name="s")

    def body(vhalves_hbm, adjs_hbm, rel_hbm, zeros_hbm, out_hbm,
             shared, srcb, dstb, relb, sidxb, rows, semi, semg, sems):
        c = lax.axis_index("c")
        s = lax.axis_index("s")
        base = s * EPT
        vsrc = vhalves_hbm.at[c]

        def issue_idx(j, b):
            off = base + j * CH
            pltpu.async_copy(adjs_hbm.at[0, pl.ds(off, CH)], srcb.at[b],
                             semi.at[b])
            pltpu.async_copy(adjs_hbm.at[1, pl.ds(off, CH)], dstb.at[b],
                             semi.at[b])
            pltpu.async_copy(rel_hbm.at[pl.ds(off, CH)], relb.at[b],
                             semi.at[b])

        def wait_idx(j, b):
            off = base + j * CH
            pltpu.make_async_copy(adjs_hbm.at[0, pl.ds(off, CH)], srcb.at[b],
                                  semi.at[b]).wait()
            pltpu.make_async_copy(adjs_hbm.at[1, pl.ds(off, CH)], dstb.at[b],
                                  semi.at[b]).wait()
            pltpu.make_async_copy(rel_hbm.at[pl.ds(off, CH)], relb.at[b],
                                  semi.at[b]).wait()

        def issue_gather(b):
            pltpu.async_copy(vsrc.at[srcb.at[b]], rows.at[b], semg.at[b])

        def wait_gather(b):
            pltpu.make_async_copy(vsrc.at[srcb.at[b]], rows.at[b],
                                  semg.at[b]).wait()

        def wait_scatter(b):
            pltpu.make_async_copy(rows.at[b], shared.at[sidxb.at[b]],
                                  sems.at[b]).wait()

        pltpu.sync_copy(zeros_hbm.at[pl.ds(s * ZSTRIPE, ZSTRIPE)],
                        shared.at[pl.ds(s * ZSTRIPE, ZSTRIPE)])

        issue_idx(0, 0)
        wait_idx(0, 0)
        issue_gather(0)
        issue_idx(1, 1)

        plsc.subcore_barrier()

        @pl.loop(0, NCHUNK, step=2)
        def _edges(j0):
            for p in range(2):
                j = j0 + p
                n = 1 - p

                @pl.when(j + 1 < NCHUNK)
                def _start_next():
                    wait_idx(j + 1, n)

                    @pl.when(j >= 1)
                    def _recycle():
                        wait_scatter(n)

                    issue_gather(n)

                wait_gather(p)
                for k in range(CH // L):
                    dv = dstb[p, pl.ds(k * L, L)]
                    rv = relb[p, pl.ds(k * L, L)]
                    sidxb[p, pl.ds(k * L, L)] = rv * N_NODES + dv
                pltpu.async_copy(rows.at[p], shared.at[sidxb.at[p]],
                                 sems.at[p], add=True)

                @pl.when(j + 2 < NCHUNK)
                def _prefetch_idx():
                    issue_idx(j + 2, p)

        wait_scatter(0)
        wait_scatter(1)

        plsc.subcore_barrier()
        pltpu.sync_copy(
            shared.at[pl.ds(s * ZSTRIPE, ZSTRIPE)],
            out_hbm.at[pl.ds(s * ZSTRIPE, ZSTRIPE), pl.ds(c * HALF, HALF)])

    run = pl.kernel(
        body,
        out_type=jax.ShapeDtypeStruct((SEGS, D_IN), jnp.float32),
        mesh=mesh,
        scratch_types=[
            pltpu.VMEM_SHARED((SEGS, HALF), jnp.float32),
            pltpu.VMEM((2, CH), jnp.int32),
            pltpu.VMEM((2, CH), jnp.int32),
            pltpu.VMEM((2, CH), jnp.int32),
            pltpu.VMEM((2, CH), jnp.int32),
            pltpu.VMEM((2, CH, HALF), jnp.float32),
            pltpu.SemaphoreType.DMA((2,)),
            pltpu.SemaphoreType.DMA((2,)),
            pltpu.SemaphoreType.DMA((2,)),
        ],
        compiler_params=pltpu.CompilerParams(use_tc_tiling_on_sc=False),
        name="dssconv_sc_aggregate",
    )
    return run(vhalves, adjs, rels, zeros)


def _tc_prep(vfts):
    TILE = 2000
    ZB = (SEGS // 2) // (N_NODES // TILE)

    def body(x_ref, o_ref, z_ref):
        xr = x_ref[...].reshape(TILE // 2, 2 * D_IN)
        o_ref[0] = jnp.concatenate(
            [xr[:, 0:HALF], xr[:, D_IN:D_IN + HALF]], axis=1)
        o_ref[1] = jnp.concatenate(
            [xr[:, HALF:D_IN], xr[:, D_IN + HALF:2 * D_IN]], axis=1)
        z_ref[...] = jnp.zeros((ZB, D_IN), jnp.float32)

    vh, zimg = pl.pallas_call(
        body,
        grid=(N_NODES // TILE,),
        in_specs=[pl.BlockSpec((TILE, D_IN), lambda i: (i, 0))],
        out_specs=[
            pl.BlockSpec((NC, TILE // 2, D_IN), lambda i: (0, i, 0)),
            pl.BlockSpec((ZB, D_IN), lambda i: (i, 0)),
        ],
        out_shape=[
            jax.ShapeDtypeStruct((NC, N_NODES // 2, D_IN), jnp.float32),
            jax.ShapeDtypeStruct((SEGS // 2, D_IN), jnp.float32),
        ],
    )(vfts)
    return vh.reshape(NC, N_NODES, HALF), zimg.reshape(SEGS, HALF)


MTILE = 2000


def _tc_body(x_ref, a0_ref, a1_ref, a2_ref,
             w10_ref, b10_ref, w12_ref, b12_ref,
             w20_ref, b20_ref, w22_ref, b22_ref, out_ref):
    x = x_ref[...]
    aggs = (a0_ref[...], a1_ref[...], a2_ref[...])
    agg_all = aggs[0] + aggs[1] + aggs[2]
    w10, b10 = w10_ref[...], b10_ref[...]
    w12, b12 = w12_ref[...], b12_ref[...]
    w20, b20 = w20_ref[...], b20_ref[...]
    w22, b22 = w22_ref[...], b22_ref[...]
    for r in range(N_REL):
        h1 = jnp.maximum(jnp.dot(x + aggs[r], w10) + b10, 0.0)
        o1 = jnp.dot(h1, w12) + b12
        h2 = jnp.maximum(jnp.dot(x + (agg_all - aggs[r]), w20) + b20, 0.0)
        o2 = jnp.dot(h2, w22) + b22
        out_ref[r] = o1 + o2


def _tc_mlp(vfts, agg, W1_0, b1_0, W1_2, b1_2, W2_0, b2_0, W2_2, b2_2):
    nb = N_NODES // MTILE

    def aspec(r):
        off = r * nb
        return pl.BlockSpec((MTILE, D_IN), lambda i, off=off: (off + i, 0))

    full = lambda shape: pl.BlockSpec(shape, lambda i: (0,) * len(shape))
    out = pl.pallas_call(
        _tc_body,
        grid=(nb,),
        in_specs=[
            pl.BlockSpec((MTILE, D_IN), lambda i: (i, 0)),
            aspec(0), aspec(1), aspec(2),
            full((D_IN, D_OUT)), full((1, D_OUT)),
            full((D_OUT, D_OUT)), full((1, D_OUT)),
            full((D_IN, D_OUT)), full((1, D_OUT)),
            full((D_OUT, D_OUT)), full((1, D_OUT)),
        ],
        out_specs=pl.BlockSpec((N_REL, MTILE, D_OUT), lambda i: (0, i, 0)),
        out_shape=jax.ShapeDtypeStruct((N_REL, N_NODES, D_OUT), jnp.float32),
    )(vfts, agg, agg, agg,
      W1_0, b1_0.reshape(1, -1), W1_2, b1_2.reshape(1, -1),
      W2_0, b2_0.reshape(1, -1), W2_2, b2_2.reshape(1, -1))
    return jnp.transpose(out, (1, 0, 2))


def kernel(vfts, adjs, rels, W1_0, b1_0, W1_2, b1_2, W2_0, b2_0, W2_2, b2_2):
    vhalves, zeros = _tc_prep(vfts)
    agg = _sc_aggregate(vhalves, adjs.astype(jnp.int32),
                        rels.astype(jnp.int32), zeros)
    return _tc_mlp(vfts, agg, W1_0, b1_0, W1_2, b1_2,
                   W2_0, b2_0, W2_2, b2_2)
